# baseline (speedup 1.0000x reference)
import jax
import jax.numpy as jnp
from jax.experimental import pallas as pl
from jax.experimental.pallas import tpu as pltpu

_ROWS = 1000000
_DIM = 64
_NCHUNKS = 8
_CHUNK = _ROWS // _NCHUNKS


def _copy_kernel(in_hbm, out_hbm, sems):
    for c in range(_NCHUNKS):
        pltpu.make_async_copy(
            in_hbm.at[pl.ds(c * _CHUNK, _CHUNK), :],
            out_hbm.at[pl.ds(c * _CHUNK, _CHUNK), :],
            sems.at[c],
        ).start()
    for c in range(_NCHUNKS):
        pltpu.make_async_copy(
            in_hbm.at[pl.ds(c * _CHUNK, _CHUNK), :],
            out_hbm.at[pl.ds(c * _CHUNK, _CHUNK), :],
            sems.at[c],
        ).wait()


def kernel(weight):
    return pl.pallas_call(
        _copy_kernel,
        in_specs=[pl.BlockSpec(memory_space=pl.ANY)],
        out_specs=pl.BlockSpec(memory_space=pl.ANY),
        out_shape=jax.ShapeDtypeStruct((_ROWS, _DIM), jnp.float32),
        scratch_shapes=[pltpu.SemaphoreType.DMA((_NCHUNKS,))],
    )(weight)


# speedup vs baseline: 1.7931x; 1.7931x over previous
import jax
import jax.numpy as jnp
from jax.experimental import pallas as pl
from jax.experimental.pallas import tpu as pltpu

_ROWS = 1000000
_DIM = 64
_WIDE_ROWS = 8000
_WIDE_COLS = _ROWS * _DIM // _WIDE_ROWS
_NCHUNKS = 8
_CHUNK = _WIDE_ROWS // _NCHUNKS


def _copy_kernel(in_hbm, out_hbm, sems):
    for c in range(_NCHUNKS):
        pltpu.make_async_copy(
            in_hbm.at[pl.ds(c * _CHUNK, _CHUNK), :],
            out_hbm.at[pl.ds(c * _CHUNK, _CHUNK), :],
            sems.at[c],
        ).start()
    for c in range(_NCHUNKS):
        pltpu.make_async_copy(
            in_hbm.at[pl.ds(c * _CHUNK, _CHUNK), :],
            out_hbm.at[pl.ds(c * _CHUNK, _CHUNK), :],
            sems.at[c],
        ).wait()


def kernel(weight):
    wide = weight.reshape(_WIDE_ROWS, _WIDE_COLS)
    out = pl.pallas_call(
        _copy_kernel,
        in_specs=[pl.BlockSpec(memory_space=pl.ANY)],
        out_specs=pl.BlockSpec(memory_space=pl.ANY),
        out_shape=jax.ShapeDtypeStruct((_WIDE_ROWS, _WIDE_COLS), jnp.float32),
        scratch_shapes=[pltpu.SemaphoreType.DMA((_NCHUNKS,))],
    )(wide)
    return out.reshape(_ROWS, _DIM)


# speedup vs baseline: 11.7918x; 6.5762x over previous
import jax
import jax.numpy as jnp
from jax.experimental import pallas as pl
from jax.experimental.pallas import tpu as pltpu

_ROWS = 1000000
_DIM = 64
_WIDE_ROWS = 8000
_WIDE_COLS = _ROWS * _DIM // _WIDE_ROWS
_BLOCK_ROWS = 320


def _copy_kernel(in_ref, out_ref):
    out_ref[...] = in_ref[...]


def kernel(weight):
    wide = weight.reshape(_WIDE_ROWS, _WIDE_COLS)
    out = pl.pallas_call(
        _copy_kernel,
        grid=(_WIDE_ROWS // _BLOCK_ROWS,),
        in_specs=[pl.BlockSpec((_BLOCK_ROWS, _WIDE_COLS), lambda i: (i, 0))],
        out_specs=pl.BlockSpec((_BLOCK_ROWS, _WIDE_COLS), lambda i: (i, 0)),
        out_shape=jax.ShapeDtypeStruct((_WIDE_ROWS, _WIDE_COLS), jnp.float32),
        compiler_params=pltpu.CompilerParams(
            dimension_semantics=("arbitrary",),
        ),
    )(wide)
    return out.reshape(_ROWS, _DIM)


# speedup vs baseline: 16.1345x; 1.3683x over previous
import jax
import jax.numpy as jnp
from jax.experimental import pallas as pl
from jax.experimental.pallas import tpu as pltpu

_ROWS = 1000000
_DIM = 64
_CHUNK_ROWS = 10000
_NCHUNKS = _ROWS // _CHUNK_ROWS
_NBUF = 12


def _in_copy(in_hbm, buf, in_sems, c, b):
    return pltpu.make_async_copy(
        in_hbm.at[pl.ds(c * _CHUNK_ROWS, _CHUNK_ROWS), :],
        buf.at[b],
        in_sems.at[b],
    )


def _out_copy(out_hbm, buf, out_sems, c, b):
    return pltpu.make_async_copy(
        buf.at[b],
        out_hbm.at[pl.ds(c * _CHUNK_ROWS, _CHUNK_ROWS), :],
        out_sems.at[b],
    )


_DRAIN = 6


def _stream_kernel(in_hbm, out_hbm, buf, in_sems, out_sems):
    for c in range(min(_NBUF - _DRAIN, _NCHUNKS)):
        _in_copy(in_hbm, buf, in_sems, c, c % _NBUF).start()
    for c in range(_NCHUNKS):
        b = c % _NBUF
        _in_copy(in_hbm, buf, in_sems, c, b).wait()
        _out_copy(out_hbm, buf, out_sems, c, b).start()
        j = c - _DRAIN
        if j >= 0:
            _out_copy(out_hbm, buf, out_sems, j, j % _NBUF).wait()
        nxt = c + _NBUF - _DRAIN
        if nxt < _NCHUNKS and nxt >= _NBUF - _DRAIN:
            _in_copy(in_hbm, buf, in_sems, nxt, nxt % _NBUF).start()
    for j in range(max(0, _NCHUNKS - _DRAIN), _NCHUNKS):
        _out_copy(out_hbm, buf, out_sems, j, j % _NBUF).wait()


def kernel(weight):
    return pl.pallas_call(
        _stream_kernel,
        in_specs=[pl.BlockSpec(memory_space=pl.ANY)],
        out_specs=pl.BlockSpec(memory_space=pl.ANY),
        out_shape=jax.ShapeDtypeStruct((_ROWS, _DIM), jnp.float32),
        scratch_shapes=[
            pltpu.VMEM((_NBUF, _CHUNK_ROWS, _DIM), jnp.float32),
            pltpu.SemaphoreType.DMA((_NBUF,)),
            pltpu.SemaphoreType.DMA((_NBUF,)),
        ],
    )(weight)
